# baseline (speedup 1.0000x reference)
import functools
import numpy as np
import jax
import jax.numpy as jnp
from jax import lax
from jax.experimental import pallas as pl
from jax.experimental.pallas import tpu as pltpu
from jax.experimental.pallas import tpu_sc as plsc

_ALPHA = 4


def _slow_idx(t: int) -> list:
    n = t // _ALPHA
    return [int(v) for v in np.linspace(0.0, t - 1, n).astype(np.int32)]


def kernel(frames):
    c, t, h, w = frames.shape
    idx = _slow_idx(t)
    n = len(idx)
    assert idx == [(g * (t - 1)) // (n - 1) for g in range(n)]

    mesh = plsc.VectorSubcoreMesh(core_axis_name="c", subcore_axis_name="s")

    @functools.partial(
        pl.kernel,
        mesh=mesh,
        out_type=jax.ShapeDtypeStruct((c, n, h, w), frames.dtype),
        scratch_types=[
            pltpu.VMEM((h, w), frames.dtype),
            pltpu.SemaphoreType.DMA,
        ],
    )
    def sc_gather(frames_hbm, slow_hbm, buf, sem):
        cid = lax.axis_index("c")
        sid = lax.axis_index("s")
        wid = sid * 2 + cid

        @pl.when(wid < c * n)
        def _():
            ch = wid // n
            j = lax.rem(wid, n)
            src_t = (j * (t - 1)) // (n - 1)
            pltpu.async_copy(frames_hbm.at[ch, src_t], buf, sem).wait()
            pltpu.async_copy(buf, slow_hbm.at[ch, j], sem).wait()

    slow = sc_gather(frames)
    return (slow, frames)


# speedup vs baseline: 1.1268x; 1.1268x over previous
import functools
import numpy as np
import jax
import jax.numpy as jnp
from jax import lax
from jax.experimental import pallas as pl
from jax.experimental.pallas import tpu as pltpu
from jax.experimental.pallas import tpu_sc as plsc

_ALPHA = 4


def _slow_idx(t: int) -> list:
    n = t // _ALPHA
    return [int(v) for v in np.linspace(0.0, t - 1, n).astype(np.int32)]


def _copy_body(in_ref, out_ref):
    out_ref[...] = in_ref[...]


def kernel(frames):
    c, t, h, w = frames.shape
    idx = _slow_idx(t)
    n = len(idx)
    assert idx == [(g * (t - 1)) // (n - 1) for g in range(n)]

    mesh = plsc.VectorSubcoreMesh(core_axis_name="c", subcore_axis_name="s")

    @functools.partial(
        pl.kernel,
        mesh=mesh,
        out_type=jax.ShapeDtypeStruct((c, n, h, w), frames.dtype),
        scratch_types=[
            pltpu.VMEM((h, w), frames.dtype),
            pltpu.SemaphoreType.DMA,
        ],
    )
    def sc_gather(frames_hbm, slow_hbm, buf, sem):
        cid = lax.axis_index("c")
        sid = lax.axis_index("s")
        wid = sid * 2 + cid

        @pl.when(wid < c * n)
        def _():
            ch = wid // n
            j = lax.rem(wid, n)
            src_t = (j * (t - 1)) // (n - 1)
            pltpu.async_copy(frames_hbm.at[ch, src_t], buf, sem).wait()
            pltpu.async_copy(buf, slow_hbm.at[ch, j], sem).wait()

    slow = sc_gather(frames)

    g_sz = 8
    fast = pl.pallas_call(
        _copy_body,
        grid=(c * t // g_sz,),
        in_specs=[
            pl.BlockSpec(
                (1, g_sz, h, w), lambda j, _n=t // g_sz: (j // _n, j % _n, 0, 0)
            )
        ],
        out_specs=pl.BlockSpec(
            (1, g_sz, h, w), lambda j, _n=t // g_sz: (j // _n, j % _n, 0, 0)
        ),
        out_shape=jax.ShapeDtypeStruct((c, t, h, w), frames.dtype),
    )(frames)
    return (slow, fast)


# speedup vs baseline: 2.1170x; 1.8787x over previous
import numpy as np
import jax
import jax.numpy as jnp
from jax.experimental import pallas as pl

_ALPHA = 4
_G = 8


def _slow_idx(t: int) -> list:
    n = t // _ALPHA
    return [int(v) for v in np.linspace(0.0, t - 1, n).astype(np.int32)]


def kernel(frames):
    c, t, h, w = frames.shape
    idx = _slow_idx(t)
    n = len(idx)
    g_sz = _G if t % _G == 0 else _ALPHA
    gpc = t // g_sz
    spg = g_sz // _ALPHA
    offs = [
        [idx[g * spg + s] - g_sz * g for s in range(spg)] for g in range(gpc)
    ]
    assert all(0 <= o < g_sz for row in offs for o in row)

    def body(in_ref, slow_ref, fast_ref):
        j = pl.program_id(0)
        g = jax.lax.rem(j, gpc)
        fast_ref[...] = in_ref[...]
        for s in range(spg):
            off = jnp.int32(offs[0][s])
            for k in range(1, gpc):
                off = jnp.where(g == k, jnp.int32(offs[k][s]), off)
            slow_ref[:, s : s + 1] = in_ref[:, pl.ds(off, 1)]

    slow, fast = pl.pallas_call(
        body,
        grid=(c * gpc,),
        in_specs=[
            pl.BlockSpec((1, g_sz, h, w), lambda j: (j // gpc, j % gpc, 0, 0))
        ],
        out_specs=[
            pl.BlockSpec((1, spg, h, w), lambda j: (j // gpc, j % gpc, 0, 0)),
            pl.BlockSpec((1, g_sz, h, w), lambda j: (j // gpc, j % gpc, 0, 0)),
        ],
        out_shape=[
            jax.ShapeDtypeStruct((c, n, h, w), frames.dtype),
            jax.ShapeDtypeStruct((c, t, h, w), frames.dtype),
        ],
    )(frames)
    return (slow, fast)


# speedup vs baseline: 2.2968x; 1.0850x over previous
import numpy as np
import jax
import jax.numpy as jnp
from jax.experimental import pallas as pl

_ALPHA = 4
_G = 16


def _slow_idx(t: int) -> list:
    n = t // _ALPHA
    return [int(v) for v in np.linspace(0.0, t - 1, n).astype(np.int32)]


def kernel(frames):
    c, t, h, w = frames.shape
    idx = _slow_idx(t)
    n = len(idx)
    g_sz = _G if t % _G == 0 else _ALPHA
    gpc = t // g_sz
    spg = g_sz // _ALPHA
    offs = [
        [idx[g * spg + s] - g_sz * g for s in range(spg)] for g in range(gpc)
    ]
    assert all(0 <= o < g_sz for row in offs for o in row)

    def body(in_ref, slow_ref, fast_ref):
        j = pl.program_id(0)
        g = jax.lax.rem(j, gpc)
        fast_ref[...] = in_ref[...]
        for s in range(spg):
            off = jnp.int32(offs[0][s])
            for k in range(1, gpc):
                off = jnp.where(g == k, jnp.int32(offs[k][s]), off)
            slow_ref[:, s : s + 1] = in_ref[:, pl.ds(off, 1)]

    slow, fast = pl.pallas_call(
        body,
        grid=(c * gpc,),
        in_specs=[
            pl.BlockSpec((1, g_sz, h, w), lambda j: (j // gpc, j % gpc, 0, 0))
        ],
        out_specs=[
            pl.BlockSpec((1, spg, h, w), lambda j: (j // gpc, j % gpc, 0, 0)),
            pl.BlockSpec((1, g_sz, h, w), lambda j: (j // gpc, j % gpc, 0, 0)),
        ],
        out_shape=[
            jax.ShapeDtypeStruct((c, n, h, w), frames.dtype),
            jax.ShapeDtypeStruct((c, t, h, w), frames.dtype),
        ],
    )(frames)
    return (slow, fast)


# speedup vs baseline: 2.6686x; 1.1619x over previous
import numpy as np
import jax
import jax.numpy as jnp
from jax.experimental import pallas as pl

_ALPHA = 4
_G = 32


def _slow_idx(t: int) -> list:
    n = t // _ALPHA
    return [int(v) for v in np.linspace(0.0, t - 1, n).astype(np.int32)]


def kernel(frames):
    c, t, h, w = frames.shape
    idx = _slow_idx(t)
    n = len(idx)
    g_sz = _G if t % _G == 0 else _ALPHA
    gpc = t // g_sz
    spg = g_sz // _ALPHA
    offs = [
        [idx[g * spg + s] - g_sz * g for s in range(spg)] for g in range(gpc)
    ]
    assert all(0 <= o < g_sz for row in offs for o in row)

    def body(in_ref, slow_ref, fast_ref):
        fast_ref[...] = in_ref[...]
        if gpc == 1:
            for s in range(spg):
                slow_ref[:, s : s + 1] = in_ref[:, offs[0][s] : offs[0][s] + 1]
        else:
            j = pl.program_id(0)
            g = jax.lax.rem(j, gpc)
            for s in range(spg):
                off = jnp.int32(offs[0][s])
                for k in range(1, gpc):
                    off = jnp.where(g == k, jnp.int32(offs[k][s]), off)
                slow_ref[:, s : s + 1] = in_ref[:, pl.ds(off, 1)]

    slow, fast = pl.pallas_call(
        body,
        grid=(c * gpc,),
        in_specs=[
            pl.BlockSpec((1, g_sz, h, w), lambda j: (j // gpc, j % gpc, 0, 0))
        ],
        out_specs=[
            pl.BlockSpec((1, spg, h, w), lambda j: (j // gpc, j % gpc, 0, 0)),
            pl.BlockSpec((1, g_sz, h, w), lambda j: (j // gpc, j % gpc, 0, 0)),
        ],
        out_shape=[
            jax.ShapeDtypeStruct((c, n, h, w), frames.dtype),
            jax.ShapeDtypeStruct((c, t, h, w), frames.dtype),
        ],
    )(frames)
    return (slow, fast)
